# baseline (speedup 1.0000x reference)
import functools

import jax
import jax.numpy as jnp
from jax import lax
from jax.experimental import pallas as pl
from jax.experimental.pallas import tpu as pltpu
from jax.experimental.pallas import tpu_sc as plsc

_MEM_ROWS = 100000
_MEM_UNIT = 256
_D_OUT = 768
_D_ALL = 1027
_CH = 2000
_NST = _MEM_ROWS // _CH
_NI = 10
_LAG = 3
_MIN_SIM = 0.5
_POS_INF = float("inf")
_NEG_INF = float("-inf")
_IMAX = 0x7FFFFFFF

_NW = 32
_RPW = _MEM_ROWS // _NW
_SCCH = 125
_SCNCH = _RPW // _SCCH
_SEG = 3136


def _matvec_body(x_ref, w_ref, b_ref, o_ref):
    o_ref[...] = jax.lax.dot_general(
        x_ref[...], w_ref[...], (((1,), (1,)), ((), ())),
        preferred_element_type=jnp.float32,
        precision=jax.lax.Precision.DEFAULT) + b_ref[...]


_sc_mesh = plsc.VectorSubcoreMesh(core_axis_name="c", subcore_axis_name="s")


_SCCH = 128
_SCFULL = 24
_SCTAIL = _RPW - _SCFULL * _SCCH


@functools.partial(
    pl.kernel,
    mesh=_sc_mesh,
    compiler_params=pltpu.CompilerParams(needs_layout_passes=False),
    out_type=[
        jax.ShapeDtypeStruct((_NW * 16,), jnp.float32),
        jax.ShapeDtypeStruct((_NW * 16,), jnp.int32),
    ],
    scratch_types=[
        pltpu.VMEM((_MEM_UNIT,), jnp.float32),
        pltpu.VMEM((_SCCH * _MEM_UNIT,), jnp.float32),
        pltpu.VMEM((_SCCH * _MEM_UNIT,), jnp.float32),
        pltpu.VMEM((16,), jnp.float32),
        pltpu.VMEM((16,), jnp.int32),
        pltpu.SemaphoreType.DMA,
        pltpu.SemaphoreType.DMA((2,)),
    ],
)
def _sc_scan(mem_hbm, m_hbm, d2_hbm, idx_hbm, mbuf, dbuf0, dbuf1, rd2, ridx,
             msem, dsem):
    wid = lax.axis_index("s") * 2 + lax.axis_index("c")
    base = wid * _RPW
    dbufs = [dbuf0, dbuf1]
    nch = _SCFULL + 1
    chrows = [_SCCH] * _SCFULL + [_SCTAIL]
    choffs = [i * _SCCH for i in range(_SCFULL)] + [_SCFULL * _SCCH]

    pltpu.make_async_copy(m_hbm, mbuf, msem).start()

    def chunk_cp(ch, b):
        off = (base + choffs[ch]) * _MEM_UNIT
        return pltpu.make_async_copy(
            mem_hbm.at[pl.ds(off, chrows[ch] * _MEM_UNIT)],
            dbufs[b].at[pl.ds(0, chrows[ch] * _MEM_UNIT)], dsem.at[b])

    chunk_cp(0, 0).start()
    chunk_cp(1, 1).start()
    pltpu.make_async_copy(m_hbm, mbuf, msem).wait()

    lane = lax.iota(jnp.int32, 16)
    best_d2 = jnp.full((16,), _POS_INF, jnp.float32)
    best_idx = jnp.full((16,), _IMAX, jnp.int32)

    for ch in range(nch):
        b = ch % 2
        chunk_cp(ch, b).wait()
        nrows = chrows[ch]
        gstarts = [g * 16 for g in range(nrows // 16)]
        if nrows % 16:
            gstarts.append(nrows - 16)
        for g0 in gstarts:
            basevec = (g0 + lane) * _MEM_UNIT

            def col_body(c, acc):
                v = plsc.load_gather(dbufs[b], [basevec + c])
                mc = plsc.load_gather(mbuf, [jnp.full((16,), c, jnp.int32)])
                dd = v - mc
                return acc + dd * dd

            acc = lax.fori_loop(0, _MEM_UNIT, col_body,
                                jnp.zeros((16,), jnp.float32))
            idx_g = base + choffs[ch] + g0 + lane
            better = acc < best_d2
            best_d2 = jnp.where(better, acc, best_d2)
            best_idx = jnp.where(better, idx_g, best_idx)
        if ch + 2 < nch:
            chunk_cp(ch + 2, b).start()

    rd2[...] = best_d2
    ridx[...] = best_idx
    pltpu.make_async_copy(rd2, d2_hbm.at[pl.ds(wid * 16, 16)], msem).start()
    pltpu.make_async_copy(rd2, d2_hbm.at[pl.ds(wid * 16, 16)], msem).wait()
    pltpu.make_async_copy(ridx, idx_hbm.at[pl.ds(wid * 16, 16)], msem).start()
    pltpu.make_async_copy(ridx, idx_hbm.at[pl.ds(wid * 16, 16)], msem).wait()


def _in_cp(mem, bufs, isem, step):
    b = step % _NI
    return pltpu.make_async_copy(
        mem.at[pl.ds(step * _CH, _CH)], bufs.at[b], isem.at[b])


def _out_cp(bufs, memo, osem, step):
    b = step % _NI
    return pltpu.make_async_copy(
        bufs.at[b], memo.at[pl.ds(step * _CH, _CH)], osem.at[b])


def _pump_body(mem_ref, memo_ref, bufs, isem, osem):
    for st in range(_NI):
        _in_cp(mem_ref, bufs, isem, st).start()
    for st in range(_NST):
        _in_cp(mem_ref, bufs, isem, st).wait()
        _out_cp(bufs, memo_ref, osem, st).start()
        st_old = st - _LAG
        if st_old >= 0 and st_old + _NI < _NST:
            _out_cp(bufs, memo_ref, osem, st_old).wait()
            _in_cp(mem_ref, bufs, isem, st_old + _NI).start()
    for st in range(_NST - _NI, _NST):
        _out_cp(bufs, memo_ref, osem, st).wait()


def _merge_body(sv_ref, hp_ref, m_ref, d2s_ref, idx_ref, mem_ref, _memo_in,
                nr_ref, memo_ref, rowb, segbuf, rsem, ssem,
                cs_ref, ci_ref):
    hp = hp_ref[0]
    s = sv_ref[0]
    j = sv_ref[1]
    w = sv_ref[2]
    m = m_ref[...]

    row_cp = pltpu.make_async_copy(mem_ref.at[pl.ds(hp, 1)], rowb, rsem)
    row_cp.start()
    row_cp.wait()
    row_new = jnp.where(w > 0.5, m, rowb[...])
    rowb[...] = row_new
    dhp = row_new - m
    d2_hp = jnp.sum(dhp * dhp)

    d2s = d2s_ref[...]
    idxs = idx_ref[...]
    contaminated = idxs == hp
    d2m = jnp.where(contaminated, _POS_INF, d2s)
    dmin = jnp.min(d2m)
    imin = jnp.min(jnp.where(d2m == dmin, idxs, _IMAX))
    has_cont = jnp.max(contaminated.astype(jnp.int32)) > 0

    cs_ref[0] = _POS_INF
    ci_ref[0] = _IMAX

    @pl.when(has_cont)
    def _rescan():
        owner = lax.div(hp, _RPW)
        seg0 = owner * _RPW
        a0 = lax.div(seg0, 8) * 8
        a0 = jnp.minimum(a0, _MEM_ROWS - _SEG)
        a0 = pl.multiple_of(a0, 8)
        seg_cp = pltpu.make_async_copy(
            mem_ref.at[pl.ds(a0, _SEG)], segbuf, ssem)
        seg_cp.start()
        seg_cp.wait()
        rows = jax.lax.broadcasted_iota(jnp.int32, (_SEG, 1), 0) + a0
        d = segbuf[...] - m
        d2 = jnp.sum(d * d, axis=1, keepdims=True)
        valid = (rows >= seg0) & (rows < seg0 + _RPW) & (rows != hp)
        d2 = jnp.where(valid, d2, _POS_INF)
        smin = jnp.min(d2)
        cs_ref[0] = smin
        ci_ref[0] = jnp.min(jnp.where(d2 == smin, rows, _IMAX))

    bd2, bix = dmin, imin
    rb = (cs_ref[0] < bd2) | ((cs_ref[0] == bd2) & (ci_ref[0] < bix))
    bd2 = jnp.where(rb, cs_ref[0], bd2)
    bix = jnp.where(rb, ci_ref[0], bix)
    hb = (d2_hp < bd2) | ((d2_hp == bd2) & (hp < bix))
    bd2 = jnp.where(hb, d2_hp, bd2)
    bix = jnp.where(hb, hp, bix)

    best_sim = 1.0 - jnp.sqrt(bd2) / _MEM_UNIT
    jumped = jnp.where(best_sim > _MIN_SIM, bix, 0)
    head0 = jnp.where(j > 0.5, jumped, hp)
    shift = (s * 3.0 - 1e-9).astype(jnp.int32) - 1
    head = jnp.mod(head0 + shift, _MEM_ROWS)

    wr_cp = pltpu.make_async_copy(rowb, memo_ref.at[pl.ds(hp, 1)], rsem)
    wr_cp.start()
    wr_cp.wait()
    rd_cp = pltpu.make_async_copy(memo_ref.at[pl.ds(head, 1)], rowb, rsem)
    rd_cp.start()
    rd_cp.wait()
    nr_ref[...] = rowb[...]


def kernel(x, prev_read, mem, W, b, head_pos):
    xj = jnp.concatenate([x, prev_read], axis=0)[None, :]
    hp = jnp.asarray(head_pos, jnp.int32).reshape(1)

    out = pl.pallas_call(
        _matvec_body,
        out_shape=jax.ShapeDtypeStruct((1, _D_ALL), jnp.float32),
    )(xj, W, b[None, :])[0]
    y = out[:_D_OUT]
    sv = out[_D_OUT:_D_OUT + 3]
    m = out[_D_OUT + 3:]

    d2s, idxs = _sc_scan(mem.reshape(-1), m)

    mem_out = pl.pallas_call(
        _pump_body,
        in_specs=[pl.BlockSpec(memory_space=pltpu.MemorySpace.HBM)],
        out_specs=pl.BlockSpec(memory_space=pltpu.MemorySpace.HBM),
        out_shape=jax.ShapeDtypeStruct((_MEM_ROWS, _MEM_UNIT), jnp.float32),
        scratch_shapes=[
            pltpu.VMEM((_NI, _CH, _MEM_UNIT), jnp.float32),
            pltpu.SemaphoreType.DMA((_NI,)),
            pltpu.SemaphoreType.DMA((_NI,)),
        ],
    )(mem)

    new_read, mem_out2 = pl.pallas_call(
        _merge_body,
        in_specs=[
            pl.BlockSpec(memory_space=pltpu.MemorySpace.SMEM),
            pl.BlockSpec(memory_space=pltpu.MemorySpace.SMEM),
            pl.BlockSpec((1, _MEM_UNIT), lambda: (0, 0)),
            pl.BlockSpec((_NW, 16), lambda: (0, 0)),
            pl.BlockSpec((_NW, 16), lambda: (0, 0)),
            pl.BlockSpec(memory_space=pltpu.MemorySpace.HBM),
            pl.BlockSpec(memory_space=pltpu.MemorySpace.HBM),
        ],
        out_specs=[
            pl.BlockSpec((1, _MEM_UNIT), lambda: (0, 0)),
            pl.BlockSpec(memory_space=pltpu.MemorySpace.HBM),
        ],
        out_shape=[
            jax.ShapeDtypeStruct((1, _MEM_UNIT), jnp.float32),
            jax.ShapeDtypeStruct((_MEM_ROWS, _MEM_UNIT), jnp.float32),
        ],
        input_output_aliases={6: 1},
        scratch_shapes=[
            pltpu.VMEM((1, _MEM_UNIT), jnp.float32),
            pltpu.VMEM((_SEG, _MEM_UNIT), jnp.float32),
            pltpu.SemaphoreType.DMA,
            pltpu.SemaphoreType.DMA,
            pltpu.SMEM((1,), jnp.float32),
            pltpu.SMEM((1,), jnp.int32),
        ],
    )(sv, hp, m[None, :], d2s.reshape(_NW, 16), idxs.reshape(_NW, 16),
      mem, mem_out)

    return (y, new_read.reshape(_MEM_UNIT), mem_out2)


# speedup vs baseline: 9.1493x; 9.1493x over previous
import jax
import jax.numpy as jnp
from jax.experimental import pallas as pl
from jax.experimental.pallas import tpu as pltpu

_MEM_ROWS = 100000
_MEM_UNIT = 256
_D_OUT = 768
_D_ALL = 1027
_CH = 1000
_NST = _MEM_ROWS // _CH
_NI = 16
_LAG = 5
_MIN_SIM = 0.5
_NEG_INF = float("-inf")
_IMAX = 0x7FFFFFFF


def _in_cp(mem, bufs, isem, step):
    b = step % _NI
    return pltpu.make_async_copy(
        mem.at[pl.ds(step * _CH, _CH)], bufs.at[b], isem.at[b])


def _out_cp(bufs, memo, osem, step):
    b = step % _NI
    return pltpu.make_async_copy(
        bufs.at[b], memo.at[pl.ds(step * _CH, _CH)], osem.at[b])


def _ntm_body(hp_ref, xj_ref, b_ref, w_hbm, mem_ref, y_ref, nr_ref, memo_ref,
              wbuf, ibufs, rowb, wsem, isem, osem, rsem,
              bs_ref, bi_ref):
    hp = hp_ref[0]

    w_cp = pltpu.make_async_copy(w_hbm, wbuf, wsem)
    w_cp.start()
    row_cp = pltpu.make_async_copy(mem_ref.at[pl.ds(hp, 1)], rowb, rsem)
    row_cp.start()
    for st in range(_NI):
        _in_cp(mem_ref, ibufs, isem, st).start()

    w_cp.wait()
    out_row = jax.lax.dot_general(
        xj_ref[...], wbuf[...], (((1,), (1,)), ((), ())),
        preferred_element_type=jnp.float32,
        precision=jax.lax.Precision.DEFAULT) + b_ref[...]
    y_ref[...] = out_row
    s = out_row[0, _D_OUT]
    j = out_row[0, _D_OUT + 1]
    w = out_row[0, _D_OUT + 2]
    m = out_row[:, _D_OUT + 3:]

    row_cp.wait()
    row_new = jnp.where(w > 0.5, m, rowb[...])
    rowb[...] = row_new
    dhp = row_new - m
    sim_hp = 1.0 - jnp.sqrt(jnp.sum(dhp * dhp)) / _MEM_UNIT

    bs_ref[0] = _NEG_INF
    bi_ref[0] = _IMAX

    for st in range(_NST):
        bi_n = st % _NI
        _in_cp(mem_ref, ibufs, isem, st).wait()
        blk = ibufs[bi_n]
        _out_cp(ibufs, memo_ref, osem, st).start()
        st_old = st - _LAG
        if st_old >= 0 and st_old + _NI < _NST:
            _out_cp(ibufs, memo_ref, osem, st_old).wait()
            _in_cp(mem_ref, ibufs, isem, st_old + _NI).start()

        rows = jax.lax.broadcasted_iota(jnp.int32, (_CH, 1), 0) + st * _CH
        d = blk - m
        d2 = jnp.sum(d * d, axis=1, keepdims=True)
        sims = 1.0 - jnp.sqrt(d2) / _MEM_UNIT
        sims = jnp.where(rows == hp, _NEG_INF, sims)
        bmax = jnp.max(sims)
        barg = jnp.min(jnp.where(sims == bmax, rows, _IMAX))

        @pl.when(bmax > bs_ref[0])
        def _upd():
            bs_ref[0] = bmax
            bi_ref[0] = barg

    for st in range(_NST - _NI, _NST):
        _out_cp(ibufs, memo_ref, osem, st).wait()

    wr_cp = pltpu.make_async_copy(rowb, memo_ref.at[pl.ds(hp, 1)], rsem)
    wr_cp.start()

    bs = bs_ref[0]
    bi = bi_ref[0]
    hp_wins = (sim_hp > bs) | ((sim_hp == bs) & (hp < bi))
    best_sim = jnp.where(hp_wins, sim_hp, bs)
    best_idx = jnp.where(hp_wins, hp, bi)
    jumped = jnp.where(best_sim > _MIN_SIM, best_idx, 0)
    head0 = jnp.where(j > 0.5, jumped, hp)
    shift = (s * 3.0 - 1e-9).astype(jnp.int32) - 1
    head = jnp.mod(head0 + shift, _MEM_ROWS)

    wr_cp.wait()
    rd_cp = pltpu.make_async_copy(memo_ref.at[pl.ds(head, 1)], rowb, rsem)
    rd_cp.start()
    rd_cp.wait()
    nr_ref[...] = rowb[...]


def kernel(x, prev_read, mem, W, b, head_pos):
    xj = jnp.concatenate([x, prev_read], axis=0)[None, :]
    hp = jnp.asarray(head_pos, jnp.int32).reshape(1)

    y2d, new_read, mem_out = pl.pallas_call(
        _ntm_body,
        in_specs=[
            pl.BlockSpec(memory_space=pltpu.MemorySpace.SMEM),
            pl.BlockSpec((1, 1024), lambda: (0, 0)),
            pl.BlockSpec((1, _D_ALL), lambda: (0, 0)),
            pl.BlockSpec(memory_space=pltpu.MemorySpace.HBM),
            pl.BlockSpec(memory_space=pltpu.MemorySpace.HBM),
        ],
        out_specs=[
            pl.BlockSpec((1, _D_ALL), lambda: (0, 0)),
            pl.BlockSpec((1, _MEM_UNIT), lambda: (0, 0)),
            pl.BlockSpec(memory_space=pltpu.MemorySpace.HBM),
        ],
        out_shape=[
            jax.ShapeDtypeStruct((1, _D_ALL), jnp.float32),
            jax.ShapeDtypeStruct((1, _MEM_UNIT), jnp.float32),
            jax.ShapeDtypeStruct((_MEM_ROWS, _MEM_UNIT), jnp.float32),
        ],
        scratch_shapes=[
            pltpu.VMEM((_D_ALL, 1024), jnp.float32),
            pltpu.VMEM((_NI, _CH, _MEM_UNIT), jnp.float32),
            pltpu.VMEM((1, _MEM_UNIT), jnp.float32),
            pltpu.SemaphoreType.DMA,
            pltpu.SemaphoreType.DMA((_NI,)),
            pltpu.SemaphoreType.DMA((_NI,)),
            pltpu.SemaphoreType.DMA,
            pltpu.SMEM((1,), jnp.float32),
            pltpu.SMEM((1,), jnp.int32),
        ],
    )(hp, xj, b[None, :], W, mem)

    return (y2d[0, :_D_OUT], new_read.reshape(_MEM_UNIT), mem_out)


# speedup vs baseline: 9.3646x; 1.0235x over previous
import jax
import jax.numpy as jnp
from jax.experimental import pallas as pl
from jax.experimental.pallas import tpu as pltpu

_MEM_ROWS = 100000
_MEM_UNIT = 256
_D_OUT = 768
_D_ALL = 1027
_CH = 4000
_NST = _MEM_ROWS // _CH
_NI = 8
_LAG = 2
_MIN_SIM = 0.5
_NEG_INF = float("-inf")
_IMAX = 0x7FFFFFFF


def _in_cp(mem, bufs, isem, step):
    b = step % _NI
    return pltpu.make_async_copy(
        mem.at[pl.ds(step * _CH, _CH)], bufs.at[b], isem.at[b])


def _out_cp(bufs, memo, osem, step):
    b = step % _NI
    return pltpu.make_async_copy(
        bufs.at[b], memo.at[pl.ds(step * _CH, _CH)], osem.at[b])


def _ntm_body(hp_ref, xj_ref, b_ref, w_hbm, mem_ref, y_ref, nr_ref, memo_ref,
              wbuf, ibufs, rowb, wsem, isem, osem, rsem,
              bs_ref, bi_ref):
    hp = hp_ref[0]

    w_cp = pltpu.make_async_copy(w_hbm, wbuf, wsem)
    w_cp.start()
    row_cp = pltpu.make_async_copy(mem_ref.at[pl.ds(hp, 1)], rowb, rsem)
    row_cp.start()
    for st in range(_NI):
        _in_cp(mem_ref, ibufs, isem, st).start()

    w_cp.wait()
    out_row = jax.lax.dot_general(
        xj_ref[...], wbuf[...], (((1,), (1,)), ((), ())),
        preferred_element_type=jnp.float32,
        precision=jax.lax.Precision.DEFAULT) + b_ref[...]
    y_ref[...] = out_row
    s = out_row[0, _D_OUT]
    j = out_row[0, _D_OUT + 1]
    w = out_row[0, _D_OUT + 2]
    m = out_row[:, _D_OUT + 3:]

    row_cp.wait()
    row_new = jnp.where(w > 0.5, m, rowb[...])
    rowb[...] = row_new
    dhp = row_new - m
    sim_hp = 1.0 - jnp.sqrt(jnp.sum(dhp * dhp)) / _MEM_UNIT

    bs_ref[0] = _NEG_INF
    bi_ref[0] = _IMAX

    for st in range(_NST):
        bi_n = st % _NI
        _in_cp(mem_ref, ibufs, isem, st).wait()
        blk = ibufs[bi_n]
        _out_cp(ibufs, memo_ref, osem, st).start()
        st_old = st - _LAG
        if st_old >= 0 and st_old + _NI < _NST:
            _out_cp(ibufs, memo_ref, osem, st_old).wait()
            _in_cp(mem_ref, ibufs, isem, st_old + _NI).start()

        rows = jax.lax.broadcasted_iota(jnp.int32, (_CH, 1), 0) + st * _CH
        d = blk - m
        d2 = jnp.sum(d * d, axis=1, keepdims=True)
        sims = 1.0 - jnp.sqrt(d2) / _MEM_UNIT
        sims = jnp.where(rows == hp, _NEG_INF, sims)
        bmax = jnp.max(sims)
        barg = jnp.min(jnp.where(sims == bmax, rows, _IMAX))

        @pl.when(bmax > bs_ref[0])
        def _upd():
            bs_ref[0] = bmax
            bi_ref[0] = barg

    for st in range(_NST - _NI, _NST):
        _out_cp(ibufs, memo_ref, osem, st).wait()

    wr_cp = pltpu.make_async_copy(rowb, memo_ref.at[pl.ds(hp, 1)], rsem)
    wr_cp.start()

    bs = bs_ref[0]
    bi = bi_ref[0]
    hp_wins = (sim_hp > bs) | ((sim_hp == bs) & (hp < bi))
    best_sim = jnp.where(hp_wins, sim_hp, bs)
    best_idx = jnp.where(hp_wins, hp, bi)
    jumped = jnp.where(best_sim > _MIN_SIM, best_idx, 0)
    head0 = jnp.where(j > 0.5, jumped, hp)
    shift = (s * 3.0 - 1e-9).astype(jnp.int32) - 1
    head = jnp.mod(head0 + shift, _MEM_ROWS)

    wr_cp.wait()
    rd_cp = pltpu.make_async_copy(memo_ref.at[pl.ds(head, 1)], rowb, rsem)
    rd_cp.start()
    rd_cp.wait()
    nr_ref[...] = rowb[...]


def kernel(x, prev_read, mem, W, b, head_pos):
    xj = jnp.concatenate([x, prev_read], axis=0)[None, :]
    hp = jnp.asarray(head_pos, jnp.int32).reshape(1)

    y2d, new_read, mem_out = pl.pallas_call(
        _ntm_body,
        in_specs=[
            pl.BlockSpec(memory_space=pltpu.MemorySpace.SMEM),
            pl.BlockSpec((1, 1024), lambda: (0, 0)),
            pl.BlockSpec((1, _D_ALL), lambda: (0, 0)),
            pl.BlockSpec(memory_space=pltpu.MemorySpace.HBM),
            pl.BlockSpec(memory_space=pltpu.MemorySpace.HBM),
        ],
        out_specs=[
            pl.BlockSpec((1, _D_ALL), lambda: (0, 0)),
            pl.BlockSpec((1, _MEM_UNIT), lambda: (0, 0)),
            pl.BlockSpec(memory_space=pltpu.MemorySpace.HBM),
        ],
        out_shape=[
            jax.ShapeDtypeStruct((1, _D_ALL), jnp.float32),
            jax.ShapeDtypeStruct((1, _MEM_UNIT), jnp.float32),
            jax.ShapeDtypeStruct((_MEM_ROWS, _MEM_UNIT), jnp.float32),
        ],
        scratch_shapes=[
            pltpu.VMEM((_D_ALL, 1024), jnp.float32),
            pltpu.VMEM((_NI, _CH, _MEM_UNIT), jnp.float32),
            pltpu.VMEM((1, _MEM_UNIT), jnp.float32),
            pltpu.SemaphoreType.DMA,
            pltpu.SemaphoreType.DMA((_NI,)),
            pltpu.SemaphoreType.DMA((_NI,)),
            pltpu.SemaphoreType.DMA,
            pltpu.SMEM((1,), jnp.float32),
            pltpu.SMEM((1,), jnp.int32),
        ],
    )(hp, xj, b[None, :], W, mem)

    return (y2d[0, :_D_OUT], new_read.reshape(_MEM_UNIT), mem_out)


# speedup vs baseline: 9.4364x; 1.0077x over previous
import jax
import jax.numpy as jnp
from jax.experimental import pallas as pl
from jax.experimental.pallas import tpu as pltpu

_MEM_ROWS = 100000
_MEM_UNIT = 256
_D_OUT = 768
_D_ALL = 1027
_CH = 5000
_NST = _MEM_ROWS // _CH
_NI = 8
_LAG = 2
_MIN_SIM = 0.5
_NEG_INF = float("-inf")
_IMAX = 0x7FFFFFFF


def _in_cp(mem, bufs, isem, step):
    b = step % _NI
    return pltpu.make_async_copy(
        mem.at[pl.ds(step * _CH, _CH)], bufs.at[b], isem.at[b])


def _out_cp(bufs, memo, osem, step):
    b = step % _NI
    return pltpu.make_async_copy(
        bufs.at[b], memo.at[pl.ds(step * _CH, _CH)], osem.at[b])


def _ntm_body(hp_ref, xj_ref, b_ref, w_hbm, mem_ref, y_ref, nr_ref, memo_ref,
              wbuf, ibufs, rowb, wsem, isem, osem, rsem,
              bs_ref, bi_ref):
    hp = hp_ref[0]

    w_cp = pltpu.make_async_copy(w_hbm, wbuf, wsem)
    w_cp.start()
    row_cp = pltpu.make_async_copy(mem_ref.at[pl.ds(hp, 1)], rowb, rsem)
    row_cp.start()
    for st in range(_NI):
        _in_cp(mem_ref, ibufs, isem, st).start()

    w_cp.wait()
    out_row = jax.lax.dot_general(
        xj_ref[...], wbuf[...], (((1,), (1,)), ((), ())),
        preferred_element_type=jnp.float32,
        precision=jax.lax.Precision.DEFAULT) + b_ref[...]
    y_ref[...] = out_row
    s = out_row[0, _D_OUT]
    j = out_row[0, _D_OUT + 1]
    w = out_row[0, _D_OUT + 2]
    m = out_row[:, _D_OUT + 3:]

    row_cp.wait()
    row_new = jnp.where(w > 0.5, m, rowb[...])
    rowb[...] = row_new
    dhp = row_new - m
    sim_hp = 1.0 - jnp.sqrt(jnp.sum(dhp * dhp)) / _MEM_UNIT

    bs_ref[0] = _NEG_INF
    bi_ref[0] = _IMAX

    for st in range(_NST):
        bi_n = st % _NI
        _in_cp(mem_ref, ibufs, isem, st).wait()
        blk = ibufs[bi_n]
        _out_cp(ibufs, memo_ref, osem, st).start()
        st_old = st - _LAG
        if st_old >= 0 and st_old + _NI < _NST:
            _out_cp(ibufs, memo_ref, osem, st_old).wait()
            _in_cp(mem_ref, ibufs, isem, st_old + _NI).start()

        rows = jax.lax.broadcasted_iota(jnp.int32, (_CH, 1), 0) + st * _CH
        d = blk - m
        d2 = jnp.sum(d * d, axis=1, keepdims=True)
        sims = 1.0 - jnp.sqrt(d2) / _MEM_UNIT
        sims = jnp.where(rows == hp, _NEG_INF, sims)
        bmax = jnp.max(sims)
        barg = jnp.min(jnp.where(sims == bmax, rows, _IMAX))

        @pl.when(bmax > bs_ref[0])
        def _upd():
            bs_ref[0] = bmax
            bi_ref[0] = barg

    for st in range(_NST - _NI, _NST):
        _out_cp(ibufs, memo_ref, osem, st).wait()

    wr_cp = pltpu.make_async_copy(rowb, memo_ref.at[pl.ds(hp, 1)], rsem)
    wr_cp.start()

    bs = bs_ref[0]
    bi = bi_ref[0]
    hp_wins = (sim_hp > bs) | ((sim_hp == bs) & (hp < bi))
    best_sim = jnp.where(hp_wins, sim_hp, bs)
    best_idx = jnp.where(hp_wins, hp, bi)
    jumped = jnp.where(best_sim > _MIN_SIM, best_idx, 0)
    head0 = jnp.where(j > 0.5, jumped, hp)
    shift = (s * 3.0 - 1e-9).astype(jnp.int32) - 1
    head = jnp.mod(head0 + shift, _MEM_ROWS)

    wr_cp.wait()
    rd_cp = pltpu.make_async_copy(memo_ref.at[pl.ds(head, 1)], rowb, rsem)
    rd_cp.start()
    rd_cp.wait()
    nr_ref[...] = rowb[...]


def kernel(x, prev_read, mem, W, b, head_pos):
    xj = jnp.concatenate([x, prev_read], axis=0)[None, :]
    hp = jnp.asarray(head_pos, jnp.int32).reshape(1)

    y2d, new_read, mem_out = pl.pallas_call(
        _ntm_body,
        in_specs=[
            pl.BlockSpec(memory_space=pltpu.MemorySpace.SMEM),
            pl.BlockSpec((1, 1024), lambda: (0, 0)),
            pl.BlockSpec((1, _D_ALL), lambda: (0, 0)),
            pl.BlockSpec(memory_space=pltpu.MemorySpace.HBM),
            pl.BlockSpec(memory_space=pltpu.MemorySpace.HBM),
        ],
        out_specs=[
            pl.BlockSpec((1, _D_ALL), lambda: (0, 0)),
            pl.BlockSpec((1, _MEM_UNIT), lambda: (0, 0)),
            pl.BlockSpec(memory_space=pltpu.MemorySpace.HBM),
        ],
        out_shape=[
            jax.ShapeDtypeStruct((1, _D_ALL), jnp.float32),
            jax.ShapeDtypeStruct((1, _MEM_UNIT), jnp.float32),
            jax.ShapeDtypeStruct((_MEM_ROWS, _MEM_UNIT), jnp.float32),
        ],
        scratch_shapes=[
            pltpu.VMEM((_D_ALL, 1024), jnp.float32),
            pltpu.VMEM((_NI, _CH, _MEM_UNIT), jnp.float32),
            pltpu.VMEM((1, _MEM_UNIT), jnp.float32),
            pltpu.SemaphoreType.DMA,
            pltpu.SemaphoreType.DMA((_NI,)),
            pltpu.SemaphoreType.DMA((_NI,)),
            pltpu.SemaphoreType.DMA,
            pltpu.SMEM((1,), jnp.float32),
            pltpu.SMEM((1,), jnp.int32),
        ],
    )(hp, xj, b[None, :], W, mem)

    return (y2d[0, :_D_OUT], new_read.reshape(_MEM_UNIT), mem_out)
